# Initial kernel scaffold; baseline (speedup 1.0000x reference)
import jax
import jax.numpy as jnp
from jax.experimental import pallas as pl


def kernel(x, edge_index, W1, b1, W2, b2):
    raise NotImplementedError("write your pallas kernel here")


# speedup vs baseline: 32.0868x; 32.0868x over previous
import functools

import jax
import jax.numpy as jnp
from jax import lax
from jax.experimental import pallas as pl
from jax.experimental.pallas import tpu as pltpu
from jax.experimental.pallas import tpu_sc as plsc

NN = 10000
NE = 320000
DF = 128
DH = 64
DO = 16

NP = 10240
EP = 327680
NC = 2
NS = 16
NW = NC * NS
EW = EP // NW
CHUNK = 128
NCH = EW // CHUNK
SPT = NP // NS

_MESH = dict(core_axis_name="c", subcore_axis_name="s",
             num_cores=NC, num_subcores=NS)


def _fill_const(ref, rows, width, value):
    vec = jnp.full((16,), value, jnp.float32)

    def body(r, _):
        for j in range(width // 16):
            ref[r, pl.ds(j * 16, 16)] = vec
        return _

    lax.fori_loop(0, rows, body, None)


@functools.partial(
    pl.kernel,
    out_type=jax.ShapeDtypeStruct((NC, NP, 16), jnp.float32),
    mesh=plsc.VectorSubcoreMesh(**_MESH),
    compiler_params=pltpu.CompilerParams(use_tc_tiling_on_sc=False),
    scratch_types=[
        pltpu.VMEM((NCH, CHUNK), jnp.int32),
        pltpu.VMEM((CHUNK, 16), jnp.float32),
        pltpu.VMEM((CHUNK, 16), jnp.float32),
        pltpu.VMEM_SHARED((NP, 16), jnp.float32),
    ],
)
def _deg_kernel(dst_hbm, out_hbm, idx_v, ones_v, zero_v, hist_sh):
    cid = lax.axis_index("c")
    sid = lax.axis_index("s")
    wid = cid * NS + sid

    _fill_const(ones_v, CHUNK, 16, 1.0)
    _fill_const(zero_v, CHUNK, 16, 0.0)

    def zero_stripe(k, _):
        pltpu.sync_copy(zero_v, hist_sh.at[pl.ds(sid * SPT + k * CHUNK, CHUNK)])
        return _

    lax.fori_loop(0, SPT // CHUNK, zero_stripe, None)
    plsc.subcore_barrier()

    pltpu.sync_copy(dst_hbm.at[wid], idx_v)

    def body(c, _):
        pltpu.sync_copy(ones_v, hist_sh.at[idx_v.at[c]], add=True)
        return _

    lax.fori_loop(0, NCH, body, None)
    plsc.subcore_barrier()

    pltpu.sync_copy(hist_sh.at[pl.ds(sid * SPT, SPT)],
                    out_hbm.at[cid, pl.ds(sid * SPT, SPT)])


def _make_agg_kernel(d):
    @functools.partial(
        pl.kernel,
        out_type=jax.ShapeDtypeStruct((NC, NP, d), jnp.float32),
        mesh=plsc.VectorSubcoreMesh(**_MESH),
        compiler_params=pltpu.CompilerParams(use_tc_tiling_on_sc=False),
        scratch_types=[
            pltpu.VMEM((NCH, CHUNK), jnp.int32),
            pltpu.VMEM((NCH, CHUNK), jnp.int32),
            pltpu.VMEM((CHUNK, d), jnp.float32),
            pltpu.VMEM((CHUNK, d), jnp.float32),
            pltpu.VMEM_SHARED((NP, d), jnp.float32),
            pltpu.SemaphoreType.DMA,
        ],
    )
    def _agg(src_hbm, dst_hbm, feat_hbm, out_hbm,
             sidx_v, didx_v, rows_v, zero_v, acc_sh, sem):
        cid = lax.axis_index("c")
        sid = lax.axis_index("s")
        wid = cid * NS + sid

        _fill_const(zero_v, CHUNK, d, 0.0)

        def zero_stripe(k, _):
            pltpu.sync_copy(zero_v,
                            acc_sh.at[pl.ds(sid * SPT + k * CHUNK, CHUNK)])
            return _

        lax.fori_loop(0, SPT // CHUNK, zero_stripe, None)
        plsc.subcore_barrier()

        pltpu.sync_copy(src_hbm.at[wid], sidx_v)
        pltpu.sync_copy(dst_hbm.at[wid], didx_v)

        def body(c, _):
            pltpu.async_copy(feat_hbm.at[sidx_v.at[c]], rows_v, sem).wait()
            pltpu.sync_copy(rows_v, acc_sh.at[didx_v.at[c]], add=True)
            return _

        lax.fori_loop(0, NCH, body, None)
        plsc.subcore_barrier()

        pltpu.sync_copy(acc_sh.at[pl.ds(sid * SPT, SPT)],
                        out_hbm.at[cid, pl.ds(sid * SPT, SPT)])

    return _agg


_agg64 = _make_agg_kernel(DH)
_agg16 = _make_agg_kernel(DO)


def _stage1_body(x_ref, w1_ref, degp_ref, hs_ref, dinv_ref):
    h = jnp.dot(x_ref[...], w1_ref[...], preferred_element_type=jnp.float32)
    deg = 1.0 + degp_ref[0, :, 0:1] + degp_ref[1, :, 0:1]
    dinv = lax.rsqrt(deg)
    hs_ref[...] = dinv * h
    dinv_ref[...] = jnp.broadcast_to(dinv, dinv_ref.shape)


def _stage2_body(aggp_ref, hs_ref, dinv_ref, b1_ref, w2_ref, zs_ref):
    dinv = dinv_ref[:, 0:1]
    p = aggp_ref[0] + aggp_ref[1] + hs_ref[...]
    h = jnp.maximum(dinv * p + b1_ref[...], 0.0)
    zs_ref[...] = dinv * jnp.dot(h, w2_ref[...],
                                 preferred_element_type=jnp.float32)


def _stage3_body(aggp_ref, zs_ref, dinv_ref, b2_ref, out_ref):
    dinv = dinv_ref[:, 0:1]
    out_ref[...] = dinv * (aggp_ref[0] + aggp_ref[1] + zs_ref[...]) + b2_ref[...]


_RB = 2048


def _stage1(xp, W1, degp):
    g = NP // _RB
    return pl.pallas_call(
        _stage1_body,
        grid=(g,),
        in_specs=[
            pl.BlockSpec((_RB, DF), lambda i: (i, 0)),
            pl.BlockSpec((DF, DH), lambda i: (0, 0)),
            pl.BlockSpec((NC, _RB, 16), lambda i: (0, i, 0)),
        ],
        out_specs=[
            pl.BlockSpec((_RB, DH), lambda i: (i, 0)),
            pl.BlockSpec((_RB, 16), lambda i: (i, 0)),
        ],
        out_shape=[
            jax.ShapeDtypeStruct((NP, DH), jnp.float32),
            jax.ShapeDtypeStruct((NP, 16), jnp.float32),
        ],
    )(xp, W1, degp)


def _stage2(agg1, hs, dinv, b1, W2p):
    g = NP // _RB
    return pl.pallas_call(
        _stage2_body,
        grid=(g,),
        in_specs=[
            pl.BlockSpec((NC, _RB, DH), lambda i: (0, i, 0)),
            pl.BlockSpec((_RB, DH), lambda i: (i, 0)),
            pl.BlockSpec((_RB, 16), lambda i: (i, 0)),
            pl.BlockSpec((1, DH), lambda i: (0, 0)),
            pl.BlockSpec((DH, DO), lambda i: (0, 0)),
        ],
        out_specs=pl.BlockSpec((_RB, DO), lambda i: (i, 0)),
        out_shape=jax.ShapeDtypeStruct((NP, DO), jnp.float32),
    )(agg1, hs, dinv, b1, W2p)


def _stage3(agg2, zs, dinv, b2p):
    g = NP // _RB
    return pl.pallas_call(
        _stage3_body,
        grid=(g,),
        in_specs=[
            pl.BlockSpec((NC, _RB, DO), lambda i: (0, i, 0)),
            pl.BlockSpec((_RB, DO), lambda i: (i, 0)),
            pl.BlockSpec((_RB, 16), lambda i: (i, 0)),
            pl.BlockSpec((1, DO), lambda i: (0, 0)),
        ],
        out_specs=pl.BlockSpec((_RB, DO), lambda i: (i, 0)),
        out_shape=jax.ShapeDtypeStruct((NP, DO), jnp.float32),
    )(agg2, zs, dinv, b2p)


def kernel(x, edge_index, W1, b1, W2, b2):
    src = edge_index[0].astype(jnp.int32)
    dst = edge_index[1].astype(jnp.int32)
    pad = NN + (jnp.arange(EP - NE, dtype=jnp.int32) % (NP - NN))
    src3 = jnp.concatenate([src, pad]).reshape(NW, NCH, CHUNK)
    dst3 = jnp.concatenate([dst, pad]).reshape(NW, NCH, CHUNK)
    xp = jnp.pad(x, ((0, NP - NN), (0, 0)))
    b1r = b1.reshape(1, DH)
    W2p = jnp.pad(W2, ((0, 0), (0, DO - W2.shape[1])))
    b2p = jnp.pad(b2, (0, DO - b2.shape[0])).reshape(1, DO)

    degp = _deg_kernel(dst3)
    hs, dinv = _stage1(xp, W1, degp)
    agg1 = _agg64(src3, dst3, hs)
    zs = _stage2(agg1, hs, dinv, b1r, W2p)
    agg2 = _agg16(src3, dst3, zs)
    outp = _stage3(agg2, zs, dinv, b2p)
    return outp[:NN, :8]


# speedup vs baseline: 49.2325x; 1.5344x over previous
import functools

import jax
import jax.numpy as jnp
from jax import lax
from jax.experimental import pallas as pl
from jax.experimental.pallas import tpu as pltpu
from jax.experimental.pallas import tpu_sc as plsc

NN = 10000
NE = 320000
DF = 128
DH = 64
DO = 16

NP = 10240
EP = 327680
NC = 2
NS = 16
NW = NC * NS
EW = EP // NW
CHUNK = 128
NCH = EW // CHUNK
SPT = NP // NS
NBUF = 4

_MESH = dict(core_axis_name="c", subcore_axis_name="s",
             num_cores=NC, num_subcores=NS)


def _fill_const(ref, rows, width, value):
    vec = jnp.full((16,), value, jnp.float32)

    def body(r, _):
        for j in range(width // 16):
            ref[r, pl.ds(j * 16, 16)] = vec
        return _

    lax.fori_loop(0, rows, body, None)


@functools.partial(
    pl.kernel,
    out_type=jax.ShapeDtypeStruct((NC, NP, 16), jnp.float32),
    mesh=plsc.VectorSubcoreMesh(**_MESH),
    compiler_params=pltpu.CompilerParams(use_tc_tiling_on_sc=False),
    scratch_types=[
        pltpu.VMEM((NCH, CHUNK), jnp.int32),
        pltpu.VMEM((CHUNK, 16), jnp.float32),
        pltpu.VMEM((CHUNK, 16), jnp.float32),
        pltpu.VMEM_SHARED((NP, 16), jnp.float32),
    ] + [pltpu.SemaphoreType.DMA] * NBUF,
)
def _deg_kernel(dst_hbm, out_hbm, idx_v, ones_v, zero_v, hist_sh, *ssem):
    cid = lax.axis_index("c")
    sid = lax.axis_index("s")
    wid = cid * NS + sid

    _fill_const(ones_v, CHUNK, 16, 1.0)
    _fill_const(zero_v, CHUNK, 16, 0.0)

    def zero_stripe(k, _):
        pltpu.sync_copy(zero_v, hist_sh.at[pl.ds(sid * SPT + k * CHUNK, CHUNK)])
        return _

    lax.fori_loop(0, SPT // CHUNK, zero_stripe, None)
    plsc.subcore_barrier()

    pltpu.sync_copy(dst_hbm.at[wid], idx_v)

    for b in range(NBUF):
        pltpu.async_copy(ones_v, hist_sh.at[idx_v.at[b]], ssem[b], add=True)

    def body(k, _):
        for b in range(NBUF):
            c = NBUF * k + b
            pltpu.make_async_copy(ones_v, hist_sh.at[idx_v.at[c]],
                                  ssem[b]).wait()
            pltpu.async_copy(ones_v, hist_sh.at[idx_v.at[c + NBUF]],
                             ssem[b], add=True)
        return _

    lax.fori_loop(0, NCH // NBUF - 1, body, None)
    for b in range(NBUF):
        c = NCH - NBUF + b
        pltpu.make_async_copy(ones_v, hist_sh.at[idx_v.at[c]], ssem[b]).wait()
    plsc.subcore_barrier()

    pltpu.sync_copy(hist_sh.at[pl.ds(sid * SPT, SPT)],
                    out_hbm.at[cid, pl.ds(sid * SPT, SPT)])


def _make_agg_kernel(d):
    @functools.partial(
        pl.kernel,
        out_type=jax.ShapeDtypeStruct((NC, NP, d), jnp.float32),
        mesh=plsc.VectorSubcoreMesh(**_MESH),
        compiler_params=pltpu.CompilerParams(use_tc_tiling_on_sc=False),
        scratch_types=[
            pltpu.VMEM((NCH, CHUNK), jnp.int32),
            pltpu.VMEM((NCH, CHUNK), jnp.int32),
            pltpu.VMEM((NBUF, CHUNK, d), jnp.float32),
            pltpu.VMEM((CHUNK, d), jnp.float32),
            pltpu.VMEM_SHARED((NP, d), jnp.float32),
        ] + [pltpu.SemaphoreType.DMA] * (2 * NBUF),
    )
    def _agg(src_hbm, dst_hbm, feat_hbm, out_hbm,
             sidx_v, didx_v, rows_v, zero_v, acc_sh, *sems):
        gsem, ssem = sems[:NBUF], sems[NBUF:]
        cid = lax.axis_index("c")
        sid = lax.axis_index("s")
        wid = cid * NS + sid

        _fill_const(zero_v, CHUNK, d, 0.0)

        def zero_stripe(k, _):
            pltpu.sync_copy(zero_v,
                            acc_sh.at[pl.ds(sid * SPT + k * CHUNK, CHUNK)])
            return _

        lax.fori_loop(0, SPT // CHUNK, zero_stripe, None)
        plsc.subcore_barrier()

        pltpu.sync_copy(src_hbm.at[wid], sidx_v)
        pltpu.sync_copy(dst_hbm.at[wid], didx_v)

        for b in range(NBUF):
            pltpu.async_copy(feat_hbm.at[sidx_v.at[b]], rows_v.at[b], gsem[b])

        def body(k, _):
            for b in range(NBUF):
                c = NBUF * k + b
                pltpu.make_async_copy(feat_hbm.at[sidx_v.at[c]],
                                      rows_v.at[b], gsem[b]).wait()
                pltpu.async_copy(rows_v.at[b], acc_sh.at[didx_v.at[c]],
                                 ssem[b], add=True)
            for b in range(NBUF):
                c = NBUF * k + b
                pltpu.make_async_copy(rows_v.at[b], acc_sh.at[didx_v.at[c]],
                                      ssem[b]).wait()
                pltpu.async_copy(feat_hbm.at[sidx_v.at[c + NBUF]],
                                 rows_v.at[b], gsem[b])
            return _

        lax.fori_loop(0, NCH // NBUF - 1, body, None)
        for b in range(NBUF):
            c = NCH - NBUF + b
            pltpu.make_async_copy(feat_hbm.at[sidx_v.at[c]],
                                  rows_v.at[b], gsem[b]).wait()
            pltpu.async_copy(rows_v.at[b], acc_sh.at[didx_v.at[c]],
                             ssem[b], add=True)
        for b in range(NBUF):
            c = NCH - NBUF + b
            pltpu.make_async_copy(rows_v.at[b], acc_sh.at[didx_v.at[c]],
                                  ssem[b]).wait()
        plsc.subcore_barrier()

        pltpu.sync_copy(acc_sh.at[pl.ds(sid * SPT, SPT)],
                        out_hbm.at[cid, pl.ds(sid * SPT, SPT)])

    return _agg


_agg64 = _make_agg_kernel(DH)
_agg16 = _make_agg_kernel(DO)


def _stage1_body(x_ref, w1_ref, degp_ref, hs_ref, dinv_ref):
    h = jnp.dot(x_ref[...], w1_ref[...], preferred_element_type=jnp.float32)
    deg = 1.0 + degp_ref[0, :, 0:1] + degp_ref[1, :, 0:1]
    dinv = lax.rsqrt(deg)
    hs_ref[...] = dinv * h
    dinv_ref[...] = jnp.broadcast_to(dinv, dinv_ref.shape)


def _stage2_body(aggp_ref, hs_ref, dinv_ref, b1_ref, w2_ref, zs_ref):
    dinv = dinv_ref[:, 0:1]
    p = aggp_ref[0] + aggp_ref[1] + hs_ref[...]
    h = jnp.maximum(dinv * p + b1_ref[...], 0.0)
    zs_ref[...] = dinv * jnp.dot(h, w2_ref[...],
                                 preferred_element_type=jnp.float32)


def _stage3_body(aggp_ref, zs_ref, dinv_ref, b2_ref, out_ref):
    dinv = dinv_ref[:, 0:1]
    out_ref[...] = dinv * (aggp_ref[0] + aggp_ref[1] + zs_ref[...]) + b2_ref[...]


_RB = 2048


def _stage1(xp, W1, degp):
    g = NP // _RB
    return pl.pallas_call(
        _stage1_body,
        grid=(g,),
        in_specs=[
            pl.BlockSpec((_RB, DF), lambda i: (i, 0)),
            pl.BlockSpec((DF, DH), lambda i: (0, 0)),
            pl.BlockSpec((NC, _RB, 16), lambda i: (0, i, 0)),
        ],
        out_specs=[
            pl.BlockSpec((_RB, DH), lambda i: (i, 0)),
            pl.BlockSpec((_RB, 16), lambda i: (i, 0)),
        ],
        out_shape=[
            jax.ShapeDtypeStruct((NP, DH), jnp.float32),
            jax.ShapeDtypeStruct((NP, 16), jnp.float32),
        ],
    )(xp, W1, degp)


def _stage2(agg1, hs, dinv, b1, W2p):
    g = NP // _RB
    return pl.pallas_call(
        _stage2_body,
        grid=(g,),
        in_specs=[
            pl.BlockSpec((NC, _RB, DH), lambda i: (0, i, 0)),
            pl.BlockSpec((_RB, DH), lambda i: (i, 0)),
            pl.BlockSpec((_RB, 16), lambda i: (i, 0)),
            pl.BlockSpec((1, DH), lambda i: (0, 0)),
            pl.BlockSpec((DH, DO), lambda i: (0, 0)),
        ],
        out_specs=pl.BlockSpec((_RB, DO), lambda i: (i, 0)),
        out_shape=jax.ShapeDtypeStruct((NP, DO), jnp.float32),
    )(agg1, hs, dinv, b1, W2p)


def _stage3(agg2, zs, dinv, b2p):
    g = NP // _RB
    return pl.pallas_call(
        _stage3_body,
        grid=(g,),
        in_specs=[
            pl.BlockSpec((NC, _RB, DO), lambda i: (0, i, 0)),
            pl.BlockSpec((_RB, DO), lambda i: (i, 0)),
            pl.BlockSpec((_RB, 16), lambda i: (i, 0)),
            pl.BlockSpec((1, DO), lambda i: (0, 0)),
        ],
        out_specs=pl.BlockSpec((_RB, DO), lambda i: (i, 0)),
        out_shape=jax.ShapeDtypeStruct((NP, DO), jnp.float32),
    )(agg2, zs, dinv, b2p)


def kernel(x, edge_index, W1, b1, W2, b2):
    src = edge_index[0].astype(jnp.int32)
    dst = edge_index[1].astype(jnp.int32)
    pad = NN + (jnp.arange(EP - NE, dtype=jnp.int32) % (NP - NN))
    src3 = jnp.concatenate([src, pad]).reshape(NW, NCH, CHUNK)
    dst3 = jnp.concatenate([dst, pad]).reshape(NW, NCH, CHUNK)
    xp = jnp.pad(x, ((0, NP - NN), (0, 0)))
    b1r = b1.reshape(1, DH)
    W2p = jnp.pad(W2, ((0, 0), (0, DO - W2.shape[1])))
    b2p = jnp.pad(b2, (0, DO - b2.shape[0])).reshape(1, DO)

    degp = _deg_kernel(dst3)
    hs, dinv = _stage1(xp, W1, degp)
    agg1 = _agg64(src3, dst3, hs)
    zs = _stage2(agg1, hs, dinv, b1r, W2p)
    agg2 = _agg16(src3, dst3, zs)
    outp = _stage3(agg2, zs, dinv, b2p)
    return outp[:NN, :8]


# speedup vs baseline: 52.4269x; 1.0649x over previous
import functools

import jax
import jax.numpy as jnp
from jax import lax
from jax.experimental import pallas as pl
from jax.experimental.pallas import tpu as pltpu
from jax.experimental.pallas import tpu_sc as plsc

NN = 10000
NE = 320000
DF = 128
DH = 64
DO = 16

NP = 10240
NC = 2
NS = 16
NW = NC * NS
EW = NE // NW
CHUNK = 80
NCH = EW // CHUNK
SPT = NP // NS
NBUF = 5

_MESH = dict(core_axis_name="c", subcore_axis_name="s",
             num_cores=NC, num_subcores=NS)


def _fill_const(ref, rows, width, value):
    vec = jnp.full((16,), value, jnp.float32)

    def body(r, _):
        for j in range(width // 16):
            ref[r, pl.ds(j * 16, 16)] = vec
        return _

    lax.fori_loop(0, rows, body, None)


@functools.partial(
    pl.kernel,
    out_type=jax.ShapeDtypeStruct((NC, NP, 16), jnp.float32),
    mesh=plsc.VectorSubcoreMesh(**_MESH),
    compiler_params=pltpu.CompilerParams(use_tc_tiling_on_sc=False),
    scratch_types=[
        pltpu.VMEM((NCH, CHUNK), jnp.int32),
        pltpu.VMEM((CHUNK, 16), jnp.float32),
        pltpu.VMEM((CHUNK, 16), jnp.float32),
        pltpu.VMEM_SHARED((NP, 16), jnp.float32),
    ] + [pltpu.SemaphoreType.DMA] * NBUF,
)
def _deg_kernel(ei_hbm, out_hbm, idx_v, ones_v, zero_v, hist_sh, *ssem):
    cid = lax.axis_index("c")
    sid = lax.axis_index("s")
    wid = cid * NS + sid

    _fill_const(ones_v, CHUNK, 16, 1.0)
    _fill_const(zero_v, CHUNK, 16, 0.0)

    def zero_stripe(k, _):
        pltpu.sync_copy(zero_v, hist_sh.at[pl.ds(sid * SPT + k * CHUNK, CHUNK)])
        return _

    lax.fori_loop(0, SPT // CHUNK, zero_stripe, None)
    plsc.subcore_barrier()

    pltpu.sync_copy(ei_hbm.at[1, wid], idx_v)

    for b in range(NBUF):
        pltpu.async_copy(ones_v, hist_sh.at[idx_v.at[b]], ssem[b], add=True)

    def body(k, _):
        for b in range(NBUF):
            c = NBUF * k + b
            pltpu.make_async_copy(ones_v, hist_sh.at[idx_v.at[c]],
                                  ssem[b]).wait()
            pltpu.async_copy(ones_v, hist_sh.at[idx_v.at[c + NBUF]],
                             ssem[b], add=True)
        return _

    lax.fori_loop(0, NCH // NBUF - 1, body, None)
    for b in range(NBUF):
        c = NCH - NBUF + b
        pltpu.make_async_copy(ones_v, hist_sh.at[idx_v.at[c]], ssem[b]).wait()
    plsc.subcore_barrier()

    pltpu.sync_copy(hist_sh.at[pl.ds(sid * SPT, SPT)],
                    out_hbm.at[cid, pl.ds(sid * SPT, SPT)])


def _make_agg_kernel(d):
    @functools.partial(
        pl.kernel,
        out_type=jax.ShapeDtypeStruct((NC, NP, d), jnp.float32),
        mesh=plsc.VectorSubcoreMesh(**_MESH),
        compiler_params=pltpu.CompilerParams(use_tc_tiling_on_sc=False),
        scratch_types=[
            pltpu.VMEM((NCH, CHUNK), jnp.int32),
            pltpu.VMEM((NCH, CHUNK), jnp.int32),
            pltpu.VMEM((NBUF, CHUNK, d), jnp.float32),
            pltpu.VMEM((CHUNK, d), jnp.float32),
            pltpu.VMEM_SHARED((NP, d), jnp.float32),
        ] + [pltpu.SemaphoreType.DMA] * (2 * NBUF),
    )
    def _agg(ei_hbm, feat_hbm, out_hbm,
             sidx_v, didx_v, rows_v, zero_v, acc_sh, *sems):
        gsem, ssem = sems[:NBUF], sems[NBUF:]
        cid = lax.axis_index("c")
        sid = lax.axis_index("s")
        wid = cid * NS + sid

        _fill_const(zero_v, CHUNK, d, 0.0)

        def zero_stripe(k, _):
            pltpu.sync_copy(zero_v,
                            acc_sh.at[pl.ds(sid * SPT + k * CHUNK, CHUNK)])
            return _

        lax.fori_loop(0, SPT // CHUNK, zero_stripe, None)
        plsc.subcore_barrier()

        pltpu.sync_copy(ei_hbm.at[0, wid], sidx_v)
        pltpu.sync_copy(ei_hbm.at[1, wid], didx_v)

        for b in range(NBUF):
            pltpu.async_copy(feat_hbm.at[sidx_v.at[b]], rows_v.at[b], gsem[b])

        def body(k, _):
            for b in range(NBUF):
                c = NBUF * k + b
                pltpu.make_async_copy(feat_hbm.at[sidx_v.at[c]],
                                      rows_v.at[b], gsem[b]).wait()
                pltpu.async_copy(rows_v.at[b], acc_sh.at[didx_v.at[c]],
                                 ssem[b], add=True)
            for b in range(NBUF):
                c = NBUF * k + b
                pltpu.make_async_copy(rows_v.at[b], acc_sh.at[didx_v.at[c]],
                                      ssem[b]).wait()
                pltpu.async_copy(feat_hbm.at[sidx_v.at[c + NBUF]],
                                 rows_v.at[b], gsem[b])
            return _

        lax.fori_loop(0, NCH // NBUF - 1, body, None)
        for b in range(NBUF):
            c = NCH - NBUF + b
            pltpu.make_async_copy(feat_hbm.at[sidx_v.at[c]],
                                  rows_v.at[b], gsem[b]).wait()
            pltpu.async_copy(rows_v.at[b], acc_sh.at[didx_v.at[c]],
                             ssem[b], add=True)
        for b in range(NBUF):
            c = NCH - NBUF + b
            pltpu.make_async_copy(rows_v.at[b], acc_sh.at[didx_v.at[c]],
                                  ssem[b]).wait()
        plsc.subcore_barrier()

        pltpu.sync_copy(acc_sh.at[pl.ds(sid * SPT, SPT)],
                        out_hbm.at[cid, pl.ds(sid * SPT, SPT)])

    return _agg


_agg64 = _make_agg_kernel(DH)
_agg16 = _make_agg_kernel(DO)


def _stage1_body(x_ref, w1_ref, degp_ref, hs_ref, dinv_ref):
    h = jnp.dot(x_ref[...], w1_ref[...], preferred_element_type=jnp.float32)
    deg = 1.0 + degp_ref[0, :, 0:1] + degp_ref[1, :, 0:1]
    dinv = lax.rsqrt(deg)
    hs_ref[...] = dinv * h
    dinv_ref[...] = jnp.broadcast_to(dinv, dinv_ref.shape)


def _stage2_body(aggp_ref, hs_ref, dinv_ref, b1_ref, w2_ref, zs_ref):
    dinv = dinv_ref[:, 0:1]
    p = aggp_ref[0] + aggp_ref[1] + hs_ref[...]
    h = jnp.maximum(dinv * p + b1_ref[...], 0.0)
    zs_ref[...] = dinv * jnp.dot(h, w2_ref[...],
                                 preferred_element_type=jnp.float32)


def _stage3_body(aggp_ref, zs_ref, dinv_ref, b2_ref, out_ref):
    dinv = dinv_ref[:, 0:1]
    q = aggp_ref[0, :, 0:8] + aggp_ref[1, :, 0:8] + zs_ref[:, 0:8]
    out_ref[...] = dinv * q + b2_ref[...]


_RB = 2000


def _stage1(x, W1, degp):
    g = NN // _RB
    return pl.pallas_call(
        _stage1_body,
        grid=(g,),
        in_specs=[
            pl.BlockSpec((_RB, DF), lambda i: (i, 0)),
            pl.BlockSpec((DF, DH), lambda i: (0, 0)),
            pl.BlockSpec((NC, _RB, 16), lambda i: (0, i, 0)),
        ],
        out_specs=[
            pl.BlockSpec((_RB, DH), lambda i: (i, 0)),
            pl.BlockSpec((_RB, 16), lambda i: (i, 0)),
        ],
        out_shape=[
            jax.ShapeDtypeStruct((NN, DH), jnp.float32),
            jax.ShapeDtypeStruct((NN, 16), jnp.float32),
        ],
    )(x, W1, degp)


def _stage2(agg1, hs, dinv, b1, W2p):
    g = NN // _RB
    return pl.pallas_call(
        _stage2_body,
        grid=(g,),
        in_specs=[
            pl.BlockSpec((NC, _RB, DH), lambda i: (0, i, 0)),
            pl.BlockSpec((_RB, DH), lambda i: (i, 0)),
            pl.BlockSpec((_RB, 16), lambda i: (i, 0)),
            pl.BlockSpec((1, DH), lambda i: (0, 0)),
            pl.BlockSpec((DH, DO), lambda i: (0, 0)),
        ],
        out_specs=pl.BlockSpec((_RB, DO), lambda i: (i, 0)),
        out_shape=jax.ShapeDtypeStruct((NN, DO), jnp.float32),
    )(agg1, hs, dinv, b1, W2p)


def _stage3(agg2, zs, dinv, b2):
    g = NN // _RB
    return pl.pallas_call(
        _stage3_body,
        grid=(g,),
        in_specs=[
            pl.BlockSpec((NC, _RB, DO), lambda i: (0, i, 0)),
            pl.BlockSpec((_RB, DO), lambda i: (i, 0)),
            pl.BlockSpec((_RB, 16), lambda i: (i, 0)),
            pl.BlockSpec((1, 8), lambda i: (0, 0)),
        ],
        out_specs=pl.BlockSpec((_RB, 8), lambda i: (i, 0)),
        out_shape=jax.ShapeDtypeStruct((NN, 8), jnp.float32),
    )(agg2, zs, dinv, b2)


def kernel(x, edge_index, W1, b1, W2, b2):
    ei3 = edge_index.astype(jnp.int32).reshape(2, NW, NCH, CHUNK)
    b1r = b1.reshape(1, DH)
    W2p = jnp.pad(W2, ((0, 0), (0, DO - W2.shape[1])))
    b2r = b2.reshape(1, 8)

    degp = _deg_kernel(ei3)
    hs, dinv = _stage1(x, W1, degp)
    agg1 = _agg64(ei3, hs)
    zs = _stage2(agg1, hs, dinv, b1r, W2p)
    agg2 = _agg16(ei3, zs)
    return _stage3(agg2, zs, dinv, b2r)


# speedup vs baseline: 56.7110x; 1.0817x over previous
import functools

import jax
import jax.numpy as jnp
from jax import lax
from jax.experimental import pallas as pl
from jax.experimental.pallas import tpu as pltpu
from jax.experimental.pallas import tpu_sc as plsc

NN = 10000
NE = 320000
DF = 128
DH = 64
DO = 16

NP = 10240
NC = 2
NS = 16
NW = NC * NS
EW = NE // NW
CHUNK = 80
NCH = EW // CHUNK
SPT = NP // NS
NBUF = 5

_MESH = dict(core_axis_name="c", subcore_axis_name="s",
             num_cores=NC, num_subcores=NS)


def _fill_const(ref, rows, width, value):
    vec = jnp.full((16,), value, jnp.float32)

    def body(r, _):
        for j in range(width // 16):
            ref[r, pl.ds(j * 16, 16)] = vec
        return _

    lax.fori_loop(0, rows, body, None)


def _fill_const_1d(ref, n, value):
    vec = jnp.full((16,), value, jnp.float32)

    def body(r, _):
        ref[pl.ds(r * 16, 16)] = vec
        return _

    lax.fori_loop(0, n // 16, body, None)


@functools.partial(
    pl.kernel,
    out_type=jax.ShapeDtypeStruct((NC, NP), jnp.float32),
    mesh=plsc.VectorSubcoreMesh(**_MESH),
    compiler_params=pltpu.CompilerParams(use_tc_tiling_on_sc=False),
    scratch_types=[
        pltpu.VMEM((NCH, CHUNK), jnp.int32),
        pltpu.VMEM((CHUNK,), jnp.float32),
        pltpu.VMEM((SPT,), jnp.float32),
        pltpu.VMEM_SHARED((NP,), jnp.float32),
    ] + [pltpu.SemaphoreType.DMA] * NBUF,
)
def _deg_kernel(ei_hbm, out_hbm, idx_v, ones_v, zero_v, hist_sh, *ssem):
    cid = lax.axis_index("c")
    sid = lax.axis_index("s")
    wid = cid * NS + sid

    _fill_const_1d(ones_v, CHUNK, 1.0)
    _fill_const_1d(zero_v, SPT, 0.0)

    pltpu.sync_copy(zero_v, hist_sh.at[pl.ds(sid * SPT, SPT)])
    plsc.subcore_barrier()

    pltpu.sync_copy(ei_hbm.at[1, wid], idx_v)

    for b in range(NBUF):
        pltpu.async_copy(ones_v, hist_sh.at[idx_v.at[b]], ssem[b], add=True)

    def body(k, _):
        for b in range(NBUF):
            c = NBUF * k + b
            pltpu.make_async_copy(ones_v, hist_sh.at[idx_v.at[c]],
                                  ssem[b]).wait()
            pltpu.async_copy(ones_v, hist_sh.at[idx_v.at[c + NBUF]],
                             ssem[b], add=True)
        return _

    lax.fori_loop(0, NCH // NBUF - 1, body, None)
    for b in range(NBUF):
        c = NCH - NBUF + b
        pltpu.make_async_copy(ones_v, hist_sh.at[idx_v.at[c]], ssem[b]).wait()
    plsc.subcore_barrier()

    pltpu.sync_copy(hist_sh.at[pl.ds(sid * SPT, SPT)],
                    out_hbm.at[cid, pl.ds(sid * SPT, SPT)])


def _make_agg_kernel(d):
    @functools.partial(
        pl.kernel,
        out_type=jax.ShapeDtypeStruct((NC, NP, d), jnp.float32),
        mesh=plsc.VectorSubcoreMesh(**_MESH),
        compiler_params=pltpu.CompilerParams(use_tc_tiling_on_sc=False),
        scratch_types=[
            pltpu.VMEM((NCH, CHUNK), jnp.int32),
            pltpu.VMEM((NCH, CHUNK), jnp.int32),
            pltpu.VMEM((NBUF, CHUNK, d), jnp.float32),
            pltpu.VMEM((CHUNK, d), jnp.float32),
            pltpu.VMEM_SHARED((NP, d), jnp.float32),
        ] + [pltpu.SemaphoreType.DMA] * (2 * NBUF),
    )
    def _agg(ei_hbm, feat_hbm, out_hbm,
             sidx_v, didx_v, rows_v, zero_v, acc_sh, *sems):
        gsem, ssem = sems[:NBUF], sems[NBUF:]
        cid = lax.axis_index("c")
        sid = lax.axis_index("s")
        wid = cid * NS + sid

        _fill_const(zero_v, CHUNK, d, 0.0)

        def zero_stripe(k, _):
            pltpu.sync_copy(zero_v,
                            acc_sh.at[pl.ds(sid * SPT + k * CHUNK, CHUNK)])
            return _

        lax.fori_loop(0, SPT // CHUNK, zero_stripe, None)
        plsc.subcore_barrier()

        pltpu.sync_copy(ei_hbm.at[0, wid], sidx_v)
        pltpu.sync_copy(ei_hbm.at[1, wid], didx_v)

        for b in range(NBUF):
            pltpu.async_copy(feat_hbm.at[sidx_v.at[b]], rows_v.at[b], gsem[b])

        def body(k, _):
            for b in range(NBUF):
                c = NBUF * k + b
                pltpu.make_async_copy(feat_hbm.at[sidx_v.at[c]],
                                      rows_v.at[b], gsem[b]).wait()
                pltpu.async_copy(rows_v.at[b], acc_sh.at[didx_v.at[c]],
                                 ssem[b], add=True)
            for b in range(NBUF):
                c = NBUF * k + b
                pltpu.make_async_copy(rows_v.at[b], acc_sh.at[didx_v.at[c]],
                                      ssem[b]).wait()
                pltpu.async_copy(feat_hbm.at[sidx_v.at[c + NBUF]],
                                 rows_v.at[b], gsem[b])
            return _

        lax.fori_loop(0, NCH // NBUF - 1, body, None)
        for b in range(NBUF):
            c = NCH - NBUF + b
            pltpu.make_async_copy(feat_hbm.at[sidx_v.at[c]],
                                  rows_v.at[b], gsem[b]).wait()
            pltpu.async_copy(rows_v.at[b], acc_sh.at[didx_v.at[c]],
                             ssem[b], add=True)
        for b in range(NBUF):
            c = NCH - NBUF + b
            pltpu.make_async_copy(rows_v.at[b], acc_sh.at[didx_v.at[c]],
                                  ssem[b]).wait()
        plsc.subcore_barrier()

        pltpu.sync_copy(acc_sh.at[pl.ds(sid * SPT, SPT)],
                        out_hbm.at[cid, pl.ds(sid * SPT, SPT)])

    return _agg


_agg64 = _make_agg_kernel(DH)
_agg16 = _make_agg_kernel(DO)


def _dinv_col(degp_ref):
    i = pl.program_id(0)
    dg = degp_ref[:, pl.ds(i * _RB, _RB)]
    deg = 1.0 + dg[0:1, :] + dg[1:2, :]
    return jnp.transpose(lax.rsqrt(deg), (1, 0))


def _stage1_body(x_ref, w1_ref, degp_ref, hs_ref):
    h = jnp.dot(x_ref[...], w1_ref[...], preferred_element_type=jnp.float32)
    hs_ref[...] = _dinv_col(degp_ref) * h


def _stage2_body(aggp_ref, hs_ref, degp_ref, b1_ref, w2_ref, zs_ref):
    dinv = _dinv_col(degp_ref)
    p = aggp_ref[0] + aggp_ref[1] + hs_ref[...]
    h = jnp.maximum(dinv * p + b1_ref[...], 0.0)
    zs_ref[...] = dinv * jnp.dot(h, w2_ref[...],
                                 preferred_element_type=jnp.float32)


def _stage3_body(aggp_ref, zs_ref, degp_ref, b2_ref, out_ref):
    q = aggp_ref[0, :, 0:8] + aggp_ref[1, :, 0:8] + zs_ref[:, 0:8]
    out_ref[...] = _dinv_col(degp_ref) * q + b2_ref[...]


_RB = 2048
_DSPEC = pl.BlockSpec((NC, NP), lambda i: (0, 0))


def _stage1(x, W1, degp):
    g = (NN + _RB - 1) // _RB
    return pl.pallas_call(
        _stage1_body,
        grid=(g,),
        in_specs=[
            pl.BlockSpec((_RB, DF), lambda i: (i, 0)),
            pl.BlockSpec((DF, DH), lambda i: (0, 0)),
            _DSPEC,
        ],
        out_specs=pl.BlockSpec((_RB, DH), lambda i: (i, 0)),
        out_shape=jax.ShapeDtypeStruct((NN, DH), jnp.float32),
    )(x, W1, degp)


def _stage2(agg1, hs, degp, b1, W2p):
    g = (NN + _RB - 1) // _RB
    return pl.pallas_call(
        _stage2_body,
        grid=(g,),
        in_specs=[
            pl.BlockSpec((NC, _RB, DH), lambda i: (0, i, 0)),
            pl.BlockSpec((_RB, DH), lambda i: (i, 0)),
            _DSPEC,
            pl.BlockSpec((1, DH), lambda i: (0, 0)),
            pl.BlockSpec((DH, DO), lambda i: (0, 0)),
        ],
        out_specs=pl.BlockSpec((_RB, DO), lambda i: (i, 0)),
        out_shape=jax.ShapeDtypeStruct((NN, DO), jnp.float32),
    )(agg1, hs, degp, b1, W2p)


def _stage3(agg2, zs, degp, b2):
    g = (NN + _RB - 1) // _RB
    return pl.pallas_call(
        _stage3_body,
        grid=(g,),
        in_specs=[
            pl.BlockSpec((NC, _RB, DO), lambda i: (0, i, 0)),
            pl.BlockSpec((_RB, DO), lambda i: (i, 0)),
            _DSPEC,
            pl.BlockSpec((1, 8), lambda i: (0, 0)),
        ],
        out_specs=pl.BlockSpec((_RB, 8), lambda i: (i, 0)),
        out_shape=jax.ShapeDtypeStruct((NN, 8), jnp.float32),
    )(agg2, zs, degp, b2)


def kernel(x, edge_index, W1, b1, W2, b2):
    ei3 = edge_index.astype(jnp.int32).reshape(2, NW, NCH, CHUNK)
    b1r = b1.reshape(1, DH)
    W2p = jnp.pad(W2, ((0, 0), (0, DO - W2.shape[1])))
    b2r = b2.reshape(1, 8)

    degp = _deg_kernel(ei3)
    hs = _stage1(x, W1, degp)
    agg1 = _agg64(ei3, hs)
    zs = _stage2(agg1, hs, degp, b1r, W2p)
    agg2 = _agg16(ei3, zs)
    return _stage3(agg2, zs, degp, b2r)


# speedup vs baseline: 57.5007x; 1.0139x over previous
import functools

import jax
import jax.numpy as jnp
from jax import lax
from jax.experimental import pallas as pl
from jax.experimental.pallas import tpu as pltpu
from jax.experimental.pallas import tpu_sc as plsc

NN = 10000
NE = 320000
DF = 128
DH = 64
DO = 16

NP = 10240
NC = 2
NS = 16
NW = NC * NS
EW = NE // NW
CHUNK = 80
NCH = EW // CHUNK
SPT = NP // NS
NBUF = 5

_MESH = dict(core_axis_name="c", subcore_axis_name="s",
             num_cores=NC, num_subcores=NS)


def _fill_const_1d(ref, n, value):
    vec = jnp.full((16,), value, jnp.float32)

    def body(r, _):
        ref[pl.ds(r * 16, 16)] = vec
        return _

    lax.fori_loop(0, n // 16, body, None)


@functools.partial(
    pl.kernel,
    out_type=jax.ShapeDtypeStruct((NC, NP), jnp.float32),
    mesh=plsc.VectorSubcoreMesh(**_MESH),
    compiler_params=pltpu.CompilerParams(use_tc_tiling_on_sc=False),
    scratch_types=[
        pltpu.VMEM((NCH, CHUNK), jnp.int32),
        pltpu.VMEM((CHUNK,), jnp.float32),
        pltpu.VMEM((SPT,), jnp.float32),
        pltpu.VMEM_SHARED((NP,), jnp.float32),
    ] + [pltpu.SemaphoreType.DMA] * NBUF,
)
def _deg_kernel(ei_hbm, out_hbm, idx_v, ones_v, zero_v, hist_sh, *ssem):
    cid = lax.axis_index("c")
    sid = lax.axis_index("s")
    wid = cid * NS + sid

    _fill_const_1d(ones_v, CHUNK, 1.0)
    _fill_const_1d(zero_v, SPT, 0.0)

    pltpu.sync_copy(zero_v, hist_sh.at[pl.ds(sid * SPT, SPT)])
    plsc.subcore_barrier()

    pltpu.sync_copy(ei_hbm.at[1, wid], idx_v)

    for b in range(NBUF):
        pltpu.async_copy(ones_v, hist_sh.at[idx_v.at[b]], ssem[b], add=True)

    def body(k, _):
        for b in range(NBUF):
            c = NBUF * k + b
            pltpu.make_async_copy(ones_v, hist_sh.at[idx_v.at[c]],
                                  ssem[b]).wait()
            pltpu.async_copy(ones_v, hist_sh.at[idx_v.at[c + NBUF]],
                             ssem[b], add=True)
        return _

    lax.fori_loop(0, NCH // NBUF - 1, body, None)
    for b in range(NBUF):
        c = NCH - NBUF + b
        pltpu.make_async_copy(ones_v, hist_sh.at[idx_v.at[c]], ssem[b]).wait()
    plsc.subcore_barrier()

    pltpu.sync_copy(hist_sh.at[pl.ds(sid * SPT, SPT)],
                    out_hbm.at[cid, pl.ds(sid * SPT, SPT)])


def _make_agg_kernel(d):
    @functools.partial(
        pl.kernel,
        out_type=jax.ShapeDtypeStruct((NC, NP, d), jnp.float32),
        mesh=plsc.VectorSubcoreMesh(**_MESH),
        compiler_params=pltpu.CompilerParams(use_tc_tiling_on_sc=False),
        scratch_types=[
            pltpu.VMEM((NCH, CHUNK), jnp.int32),
            pltpu.VMEM((NCH, CHUNK), jnp.int32),
            pltpu.VMEM((NBUF, CHUNK, d), jnp.float32),
            pltpu.VMEM_SHARED((NP, d), jnp.float32),
        ] + [pltpu.SemaphoreType.DMA] * (2 * NBUF),
    )
    def _agg(ei_hbm, feat_hbm, zin_hbm, out_hbm,
             sidx_v, didx_v, rows_v, acc_sh, *sems):
        gsem, ssem = sems[:NBUF], sems[NBUF:]
        cid = lax.axis_index("c")
        sid = lax.axis_index("s")
        wid = cid * NS + sid

        pltpu.sync_copy(zin_hbm, acc_sh.at[pl.ds(sid * SPT, SPT)])
        plsc.subcore_barrier()

        pltpu.sync_copy(ei_hbm.at[0, wid], sidx_v)
        pltpu.sync_copy(ei_hbm.at[1, wid], didx_v)

        for b in range(NBUF):
            pltpu.async_copy(feat_hbm.at[sidx_v.at[b]], rows_v.at[b], gsem[b])

        def body(k, _):
            for b in range(NBUF):
                c = NBUF * k + b
                pltpu.make_async_copy(feat_hbm.at[sidx_v.at[c]],
                                      rows_v.at[b], gsem[b]).wait()
                pltpu.async_copy(rows_v.at[b], acc_sh.at[didx_v.at[c]],
                                 ssem[b], add=True)
            for b in range(NBUF):
                c = NBUF * k + b
                pltpu.make_async_copy(rows_v.at[b], acc_sh.at[didx_v.at[c]],
                                      ssem[b]).wait()
                pltpu.async_copy(feat_hbm.at[sidx_v.at[c + NBUF]],
                                 rows_v.at[b], gsem[b])
            return _

        lax.fori_loop(0, NCH // NBUF - 1, body, None)
        for b in range(NBUF):
            c = NCH - NBUF + b
            pltpu.make_async_copy(feat_hbm.at[sidx_v.at[c]],
                                  rows_v.at[b], gsem[b]).wait()
            pltpu.async_copy(rows_v.at[b], acc_sh.at[didx_v.at[c]],
                             ssem[b], add=True)
        for b in range(NBUF):
            c = NCH - NBUF + b
            pltpu.make_async_copy(rows_v.at[b], acc_sh.at[didx_v.at[c]],
                                  ssem[b]).wait()
        plsc.subcore_barrier()

        pltpu.sync_copy(acc_sh.at[pl.ds(sid * SPT, SPT)],
                        out_hbm.at[cid, pl.ds(sid * SPT, SPT)])

    return _agg


_agg64 = _make_agg_kernel(DH)
_agg16 = _make_agg_kernel(DO)


def _dinv_col(degp_ref):
    i = pl.program_id(0)
    dg = degp_ref[:, pl.ds(i * _RB, _RB)]
    deg = 1.0 + dg[0:1, :] + dg[1:2, :]
    return jnp.transpose(lax.rsqrt(deg), (1, 0))


def _stage1_body(x_ref, w1_ref, degp_ref, hs_ref):
    h = jnp.dot(x_ref[...], w1_ref[...], preferred_element_type=jnp.float32)
    hs_ref[...] = _dinv_col(degp_ref) * h


def _stage2_body(aggp_ref, hs_ref, degp_ref, b1_ref, w2_ref, zs_ref):
    dinv = _dinv_col(degp_ref)
    p = aggp_ref[0] + aggp_ref[1] + hs_ref[...]
    h = jnp.maximum(dinv * p + b1_ref[...], 0.0)
    zs_ref[...] = dinv * jnp.dot(h, w2_ref[...],
                                 preferred_element_type=jnp.float32)


_NPR = 128 // DO


def _stage3_body(aggw_ref, zsw_ref, degp_ref, b2w_ref, out_ref):
    dinv = _dinv_col(degp_ref)
    dinvw = jnp.broadcast_to(dinv.reshape(_RB // _NPR, _NPR, 1),
                             (_RB // _NPR, _NPR, DO)).reshape(_RB // _NPR, 128)
    q = aggw_ref[0] + aggw_ref[1] + zsw_ref[...]
    out_ref[...] = dinvw * q + b2w_ref[...]


_RB = 2048
_DSPEC = pl.BlockSpec((NC, NP), lambda i: (0, 0))


def _stage1(x, W1, degp):
    g = (NN + _RB - 1) // _RB
    return pl.pallas_call(
        _stage1_body,
        grid=(g,),
        in_specs=[
            pl.BlockSpec((_RB, DF), lambda i: (i, 0)),
            pl.BlockSpec((DF, DH), lambda i: (0, 0)),
            _DSPEC,
        ],
        out_specs=pl.BlockSpec((_RB, DH), lambda i: (i, 0)),
        out_shape=jax.ShapeDtypeStruct((NN, DH), jnp.float32),
    )(x, W1, degp)


def _stage2(agg1, hs, degp, b1, W2p):
    g = (NN + _RB - 1) // _RB
    return pl.pallas_call(
        _stage2_body,
        grid=(g,),
        in_specs=[
            pl.BlockSpec((NC, _RB, DH), lambda i: (0, i, 0)),
            pl.BlockSpec((_RB, DH), lambda i: (i, 0)),
            _DSPEC,
            pl.BlockSpec((1, DH), lambda i: (0, 0)),
            pl.BlockSpec((DH, DO), lambda i: (0, 0)),
        ],
        out_specs=pl.BlockSpec((_RB, DO), lambda i: (i, 0)),
        out_shape=jax.ShapeDtypeStruct((NN, DO), jnp.float32),
    )(agg1, hs, degp, b1, W2p)


_PB = _RB // _NPR


def _stage3(agg2w, zsw, degp, b2w):
    g = (NN + _RB - 1) // _RB
    return pl.pallas_call(
        _stage3_body,
        grid=(g,),
        in_specs=[
            pl.BlockSpec((NC, _PB, 128), lambda i: (0, i, 0)),
            pl.BlockSpec((_PB, 128), lambda i: (i, 0)),
            _DSPEC,
            pl.BlockSpec((1, 128), lambda i: (0, 0)),
        ],
        out_specs=pl.BlockSpec((_PB, 128), lambda i: (i, 0)),
        out_shape=jax.ShapeDtypeStruct((NN * DO // 128, 128), jnp.float32),
    )(agg2w, zsw, degp, b2w)


def kernel(x, edge_index, W1, b1, W2, b2):
    ei3 = edge_index.astype(jnp.int32).reshape(2, NW, NCH, CHUNK)
    b1r = b1.reshape(1, DH)
    W2p = jnp.pad(W2, ((0, 0), (0, DO - W2.shape[1])))
    b2w = jnp.tile(jnp.pad(b2, (0, DO - b2.shape[0])).reshape(1, DO),
                   (1, _NPR))
    z64 = jnp.zeros((SPT, DH), jnp.float32)
    z16 = jnp.zeros((SPT, DO), jnp.float32)

    degp = _deg_kernel(ei3)
    hs = _stage1(x, W1, degp)
    agg1 = _agg64(ei3, hs, z64)
    zs = _stage2(agg1, hs, degp, b1r, W2p)
    agg2 = _agg16(ei3, zs, z16)
    agg2w = agg2.reshape(NC, NP * DO // 128, 128)
    zsw = zs.reshape(NN * DO // 128, 128)
    outw = _stage3(agg2w, zsw, degp, b2w)
    return outw.reshape(NN, DO)[:, :8]


# speedup vs baseline: 61.6482x; 1.0721x over previous
import functools

import jax
import jax.numpy as jnp
from jax import lax
from jax.experimental import pallas as pl
from jax.experimental.pallas import tpu as pltpu
from jax.experimental.pallas import tpu_sc as plsc

NN = 10000
NE = 320000
DF = 128
DH = 64
DO = 16

NP = 10240
NC = 2
NS = 16
NW = NC * NS
EW = NE // NW
CHUNK = 80
NCH = EW // CHUNK
SPT = NP // NS
NBUF = 5

_MESH = dict(core_axis_name="c", subcore_axis_name="s",
             num_cores=NC, num_subcores=NS)


def _fill_const_1d(ref, n, value):
    vec = jnp.full((16,), value, jnp.float32)

    def body(r, _):
        ref[pl.ds(r * 16, 16)] = vec
        return _

    lax.fori_loop(0, n // 16, body, None)


@functools.partial(
    pl.kernel,
    out_type=jax.ShapeDtypeStruct((NC, NP), jnp.float32),
    mesh=plsc.VectorSubcoreMesh(**_MESH),
    compiler_params=pltpu.CompilerParams(use_tc_tiling_on_sc=False),
    scratch_types=[
        pltpu.VMEM((NCH, CHUNK), jnp.int32),
        pltpu.VMEM((CHUNK,), jnp.float32),
        pltpu.VMEM((SPT,), jnp.float32),
        pltpu.VMEM_SHARED((NP,), jnp.float32),
    ] + [pltpu.SemaphoreType.DMA] * NBUF,
)
def _deg_kernel(ei_hbm, out_hbm, idx_v, ones_v, zero_v, hist_sh, *ssem):
    cid = lax.axis_index("c")
    sid = lax.axis_index("s")
    wid = cid * NS + sid

    _fill_const_1d(ones_v, CHUNK, 1.0)
    _fill_const_1d(zero_v, SPT, 0.0)

    pltpu.sync_copy(zero_v, hist_sh.at[pl.ds(sid * SPT, SPT)])
    plsc.subcore_barrier()

    pltpu.sync_copy(ei_hbm.at[1, wid], idx_v)

    for b in range(NBUF):
        pltpu.async_copy(ones_v, hist_sh.at[idx_v.at[b]], ssem[b], add=True)

    def body(k, _):
        for b in range(NBUF):
            c = NBUF * k + b
            pltpu.make_async_copy(ones_v, hist_sh.at[idx_v.at[c]],
                                  ssem[b]).wait()
            pltpu.async_copy(ones_v, hist_sh.at[idx_v.at[c + NBUF]],
                             ssem[b], add=True)
        return _

    lax.fori_loop(0, NCH // NBUF - 1, body, None)
    for b in range(NBUF):
        c = NCH - NBUF + b
        pltpu.make_async_copy(ones_v, hist_sh.at[idx_v.at[c]], ssem[b]).wait()
    plsc.subcore_barrier()

    pltpu.sync_copy(hist_sh.at[pl.ds(sid * SPT, SPT)],
                    out_hbm.at[cid, pl.ds(sid * SPT, SPT)])


def _make_agg_kernel(d):
    @functools.partial(
        pl.kernel,
        out_type=jax.ShapeDtypeStruct((NC, NP, d), jnp.float32),
        mesh=plsc.VectorSubcoreMesh(**_MESH),
        compiler_params=pltpu.CompilerParams(use_tc_tiling_on_sc=False),
        scratch_types=[
            pltpu.VMEM((NCH, CHUNK), jnp.int32),
            pltpu.VMEM((NCH, CHUNK), jnp.int32),
            pltpu.VMEM((NBUF, CHUNK, d), jnp.float32),
            pltpu.VMEM_SHARED((NP, d), jnp.float32),
        ] + [pltpu.SemaphoreType.DMA] * (2 * NBUF),
    )
    def _agg(ei_hbm, feat_hbm, zin_hbm, out_hbm,
             sidx_v, didx_v, rows_v, acc_sh, *sems):
        gsem, ssem = sems[:NBUF], sems[NBUF:]
        cid = lax.axis_index("c")
        sid = lax.axis_index("s")
        wid = cid * NS + sid

        pltpu.sync_copy(zin_hbm, acc_sh.at[pl.ds(sid * SPT, SPT)])
        plsc.subcore_barrier()

        pltpu.sync_copy(ei_hbm.at[0, wid], sidx_v)
        pltpu.sync_copy(ei_hbm.at[1, wid], didx_v)

        for b in range(NBUF):
            pltpu.async_copy(feat_hbm.at[sidx_v.at[b]], rows_v.at[b], gsem[b])

        def body(k, _):
            for b in range(NBUF):
                c = NBUF * k + b
                pltpu.make_async_copy(feat_hbm.at[sidx_v.at[c]],
                                      rows_v.at[b], gsem[b]).wait()
                pltpu.async_copy(rows_v.at[b], acc_sh.at[didx_v.at[c]],
                                 ssem[b], add=True)
            for b in range(NBUF):
                c = NBUF * k + b
                pltpu.make_async_copy(rows_v.at[b], acc_sh.at[didx_v.at[c]],
                                      ssem[b]).wait()
                pltpu.async_copy(feat_hbm.at[sidx_v.at[c + NBUF]],
                                 rows_v.at[b], gsem[b])
            return _

        lax.fori_loop(0, NCH // NBUF - 1, body, None)
        for b in range(NBUF):
            c = NCH - NBUF + b
            pltpu.make_async_copy(feat_hbm.at[sidx_v.at[c]],
                                  rows_v.at[b], gsem[b]).wait()
            pltpu.async_copy(rows_v.at[b], acc_sh.at[didx_v.at[c]],
                             ssem[b], add=True)
        for b in range(NBUF):
            c = NCH - NBUF + b
            pltpu.make_async_copy(rows_v.at[b], acc_sh.at[didx_v.at[c]],
                                  ssem[b]).wait()
        plsc.subcore_barrier()

        pltpu.sync_copy(acc_sh.at[pl.ds(sid * SPT, SPT)],
                        out_hbm.at[cid, pl.ds(sid * SPT, SPT)])

    return _agg


_agg64 = _make_agg_kernel(DH)
_agg16 = _make_agg_kernel(DO)


def _dinv_col(degp_ref):
    i = pl.program_id(0)
    dg = degp_ref[:, pl.ds(i * _RB, _RB)]
    deg = 1.0 + dg[0:1, :] + dg[1:2, :]
    return jnp.transpose(lax.rsqrt(deg), (1, 0))


def _dinv_packed(degp_ref, npr, do):
    dinv = _dinv_col(degp_ref)
    return jnp.broadcast_to(
        dinv.reshape(_RB // npr, npr, 1),
        (_RB // npr, npr, do)).reshape(_RB // npr, npr * do)


def _stage1_body(xw_ref, w1b_ref, degp_ref, hsw_ref):
    h = jnp.dot(xw_ref[...], w1b_ref[...], preferred_element_type=jnp.float32)
    hsw_ref[...] = _dinv_packed(degp_ref, 2, DH) * h


def _stage2_body(aggw_ref, hsw_ref, degp_ref, b1w_ref, w2b_ref, zs_ref):
    dinvw = _dinv_packed(degp_ref, 2, DH)
    p = aggw_ref[0] + aggw_ref[1] + hsw_ref[...]
    h = jnp.maximum(dinvw * p + b1w_ref[...], 0.0)
    z = jnp.dot(h, w2b_ref[...], preferred_element_type=jnp.float32)
    zs_ref[...] = _dinv_packed(degp_ref, 2, DO) * z


_NPR = 128 // DO


def _stage3_body(aggw_ref, zsw_ref, degp_ref, b2w_ref, out_ref):
    dinv = _dinv_col(degp_ref)
    dinvw = jnp.broadcast_to(dinv.reshape(_RB // _NPR, _NPR, 1),
                             (_RB // _NPR, _NPR, DO)).reshape(_RB // _NPR, 128)
    q = aggw_ref[0] + aggw_ref[1] + zsw_ref[...]
    out_ref[...] = dinvw * q + b2w_ref[...]


_RB = 2048
_DSPEC = pl.BlockSpec((NC, NP), lambda i: (0, 0))


_R2 = _RB // 2


def _stage1(xw, W1b, degp):
    g = (NN + _RB - 1) // _RB
    return pl.pallas_call(
        _stage1_body,
        grid=(g,),
        in_specs=[
            pl.BlockSpec((_R2, 2 * DF), lambda i: (i, 0)),
            pl.BlockSpec((2 * DF, 2 * DH), lambda i: (0, 0)),
            _DSPEC,
        ],
        out_specs=pl.BlockSpec((_R2, 2 * DH), lambda i: (i, 0)),
        out_shape=jax.ShapeDtypeStruct((NN // 2, 2 * DH), jnp.float32),
    )(xw, W1b, degp)


def _stage2(agg1w, hsw, degp, b1w, W2b):
    g = (NN + _RB - 1) // _RB
    return pl.pallas_call(
        _stage2_body,
        grid=(g,),
        in_specs=[
            pl.BlockSpec((NC, _R2, 2 * DH), lambda i: (0, i, 0)),
            pl.BlockSpec((_R2, 2 * DH), lambda i: (i, 0)),
            _DSPEC,
            pl.BlockSpec((1, 2 * DH), lambda i: (0, 0)),
            pl.BlockSpec((2 * DH, 2 * DO), lambda i: (0, 0)),
        ],
        out_specs=pl.BlockSpec((_R2, 2 * DO), lambda i: (i, 0)),
        out_shape=jax.ShapeDtypeStruct((NN // 2, 2 * DO), jnp.float32),
    )(agg1w, hsw, degp, b1w, W2b)


_PB = _RB // _NPR


def _stage3(agg2w, zsw, degp, b2w):
    g = (NN + _RB - 1) // _RB
    return pl.pallas_call(
        _stage3_body,
        grid=(g,),
        in_specs=[
            pl.BlockSpec((NC, _PB, 128), lambda i: (0, i, 0)),
            pl.BlockSpec((_PB, 128), lambda i: (i, 0)),
            _DSPEC,
            pl.BlockSpec((1, 128), lambda i: (0, 0)),
        ],
        out_specs=pl.BlockSpec((_PB, 128), lambda i: (i, 0)),
        out_shape=jax.ShapeDtypeStruct((NN * DO // 128, 128), jnp.float32),
    )(agg2w, zsw, degp, b2w)


def _blockdiag2(W):
    r, c = W.shape
    Wb = jnp.zeros((2 * r, 2 * c), W.dtype)
    return Wb.at[:r, :c].set(W).at[r:, c:].set(W)


def kernel(x, edge_index, W1, b1, W2, b2):
    ei3 = edge_index.astype(jnp.int32).reshape(2, NW, NCH, CHUNK)
    xw = x.reshape(NN // 2, 2 * DF)
    W1b = _blockdiag2(W1)
    W2p = jnp.pad(W2, ((0, 0), (0, DO - W2.shape[1])))
    W2b = _blockdiag2(W2p)
    b1w = jnp.tile(b1.reshape(1, DH), (1, 2))
    b2w = jnp.tile(jnp.pad(b2, (0, DO - b2.shape[0])).reshape(1, DO),
                   (1, _NPR))
    z64 = jnp.zeros((SPT, DH), jnp.float32)
    z16 = jnp.zeros((SPT, DO), jnp.float32)

    degp = _deg_kernel(ei3)
    hsw = _stage1(xw, W1b, degp)
    hs = hsw.reshape(NN, DH)
    agg1 = _agg64(ei3, hs, z64)
    agg1w = agg1.reshape(NC, NP * DH // 128, 128)
    zs = _stage2(agg1w, hsw, degp, b1w, W2b)
    zs16 = zs.reshape(NN, DO)
    agg2 = _agg16(ei3, zs16, z16)
    agg2w = agg2.reshape(NC, NP * DO // 128, 128)
    zsw = zs.reshape(NN * DO // 128, 128)
    outw = _stage3(agg2w, zsw, degp, b2w)
    return outw.reshape(NN, DO)[:, :8]


# speedup vs baseline: 62.7017x; 1.0171x over previous
import functools

import jax
import jax.numpy as jnp
from jax import lax
from jax.experimental import pallas as pl
from jax.experimental.pallas import tpu as pltpu
from jax.experimental.pallas import tpu_sc as plsc

NN = 10000
NE = 320000
DF = 128
DH = 64
DO = 16

NP = 10240
NC = 2
NS = 16
NW = NC * NS
EW = NE // NW
CHUNK = 80
NCH = EW // CHUNK
SPT = NP // NS
NBUF = 5

_MESH = dict(core_axis_name="c", subcore_axis_name="s",
             num_cores=NC, num_subcores=NS)


def _fill_const_1d(ref, n, value):
    vec = jnp.full((16,), value, jnp.float32)

    def body(r, _):
        ref[pl.ds(r * 16, 16)] = vec
        return _

    lax.fori_loop(0, n // 16, body, None)


@functools.partial(
    pl.kernel,
    out_type=jax.ShapeDtypeStruct((NC, NP), jnp.float32),
    mesh=plsc.VectorSubcoreMesh(**_MESH),
    compiler_params=pltpu.CompilerParams(use_tc_tiling_on_sc=False),
    scratch_types=[
        pltpu.VMEM((NCH, CHUNK), jnp.int32),
        pltpu.VMEM((CHUNK,), jnp.float32),
        pltpu.VMEM((SPT,), jnp.float32),
        pltpu.VMEM_SHARED((NP,), jnp.float32),
    ] + [pltpu.SemaphoreType.DMA] * NBUF,
)
def _deg_kernel(ei_hbm, out_hbm, idx_v, ones_v, zero_v, hist_sh, *ssem):
    cid = lax.axis_index("c")
    sid = lax.axis_index("s")
    wid = cid * NS + sid

    _fill_const_1d(ones_v, CHUNK, 1.0)
    _fill_const_1d(zero_v, SPT, 0.0)

    pltpu.sync_copy(zero_v, hist_sh.at[pl.ds(sid * SPT, SPT)])
    plsc.subcore_barrier()

    pltpu.sync_copy(ei_hbm.at[1, wid], idx_v)

    for b in range(NBUF):
        pltpu.async_copy(ones_v, hist_sh.at[idx_v.at[b]], ssem[b], add=True)

    def body(k, _):
        for b in range(NBUF):
            c = NBUF * k + b
            pltpu.make_async_copy(ones_v, hist_sh.at[idx_v.at[c]],
                                  ssem[b]).wait()
            pltpu.async_copy(ones_v, hist_sh.at[idx_v.at[c + NBUF]],
                             ssem[b], add=True)
        return _

    lax.fori_loop(0, NCH // NBUF - 1, body, None)
    for b in range(NBUF):
        c = NCH - NBUF + b
        pltpu.make_async_copy(ones_v, hist_sh.at[idx_v.at[c]], ssem[b]).wait()
    plsc.subcore_barrier()

    pltpu.sync_copy(hist_sh.at[pl.ds(sid * SPT, SPT)],
                    out_hbm.at[cid, pl.ds(sid * SPT, SPT)])


def _make_agg_kernel(d):
    @functools.partial(
        pl.kernel,
        out_type=jax.ShapeDtypeStruct((NC, NP, d), jnp.float32),
        mesh=plsc.VectorSubcoreMesh(**_MESH),
        compiler_params=pltpu.CompilerParams(use_tc_tiling_on_sc=False),
        scratch_types=[
            pltpu.VMEM((NCH, CHUNK), jnp.int32),
            pltpu.VMEM((NCH, CHUNK), jnp.int32),
            pltpu.VMEM((NBUF, CHUNK, d), jnp.float32),
            pltpu.VMEM_SHARED((NP, d), jnp.float32),
        ] + [pltpu.SemaphoreType.DMA] * (2 * NBUF),
    )
    def _agg(ei_hbm, feat_hbm, zin_hbm, out_hbm,
             sidx_v, didx_v, rows_v, acc_sh, *sems):
        gsem, ssem = sems[:NBUF], sems[NBUF:]
        cid = lax.axis_index("c")
        sid = lax.axis_index("s")
        wid = cid * NS + sid

        pltpu.sync_copy(zin_hbm, acc_sh.at[pl.ds(sid * SPT, SPT)])
        plsc.subcore_barrier()

        pltpu.sync_copy(ei_hbm.at[0, wid], sidx_v)
        pltpu.sync_copy(ei_hbm.at[1, wid], didx_v)

        for b in range(NBUF):
            pltpu.async_copy(feat_hbm.at[sidx_v.at[b]], rows_v.at[b], gsem[b])

        def body(k, _):
            for b in range(NBUF):
                c = NBUF * k + b
                pltpu.make_async_copy(feat_hbm.at[sidx_v.at[c]],
                                      rows_v.at[b], gsem[b]).wait()
                pltpu.async_copy(rows_v.at[b], acc_sh.at[didx_v.at[c]],
                                 ssem[b], add=True)
            for b in range(NBUF):
                c = NBUF * k + b
                pltpu.make_async_copy(rows_v.at[b], acc_sh.at[didx_v.at[c]],
                                      ssem[b]).wait()
                pltpu.async_copy(feat_hbm.at[sidx_v.at[c + NBUF]],
                                 rows_v.at[b], gsem[b])
            return _

        lax.fori_loop(0, NCH // NBUF - 1, body, None)
        for b in range(NBUF):
            c = NCH - NBUF + b
            pltpu.make_async_copy(feat_hbm.at[sidx_v.at[c]],
                                  rows_v.at[b], gsem[b]).wait()
            pltpu.async_copy(rows_v.at[b], acc_sh.at[didx_v.at[c]],
                             ssem[b], add=True)
        for b in range(NBUF):
            c = NCH - NBUF + b
            pltpu.make_async_copy(rows_v.at[b], acc_sh.at[didx_v.at[c]],
                                  ssem[b]).wait()
        plsc.subcore_barrier()

        pltpu.sync_copy(acc_sh.at[pl.ds(sid * SPT, SPT)],
                        out_hbm.at[cid, pl.ds(sid * SPT, SPT)])

    return _agg


_agg64 = _make_agg_kernel(DH)
_agg16 = _make_agg_kernel(DO)


def _dinv_col(degp_ref):
    i = pl.program_id(0)
    dg = degp_ref[:, pl.ds(i * _RB, _RB)]
    deg = 1.0 + dg[0:1, :] + dg[1:2, :]
    return jnp.transpose(lax.rsqrt(deg), (1, 0))


def _dinv_packed(degp_ref, npr, do):
    dinv = _dinv_col(degp_ref)
    return jnp.broadcast_to(
        dinv.reshape(_RB // npr, npr, 1),
        (_RB // npr, npr, do)).reshape(_RB // npr, npr * do)


def _stage1_body(xw_ref, w1b_ref, degp_ref, hsw_ref, dvw_ref, dv8_ref):
    h = jnp.dot(xw_ref[...], w1b_ref[...], preferred_element_type=jnp.float32)
    dinvw = _dinv_packed(degp_ref, 2, DH)
    hsw_ref[...] = dinvw * h
    dvw_ref[...] = dinvw
    dv8_ref[...] = _dinv_packed(degp_ref, _NPR, DO)


def _stage2_body(aggw_ref, hsw_ref, dvw_ref, b1w_ref, w2b_ref, zs_ref):
    dinvw = dvw_ref[...]
    p = aggw_ref[0] + aggw_ref[1] + hsw_ref[...]
    h = dinvw * jnp.maximum(dinvw * p + b1w_ref[...], 0.0)
    zs_ref[...] = jnp.dot(h, w2b_ref[...], preferred_element_type=jnp.float32)


_NPR = 128 // DO


def _stage3_body(aggw_ref, zsw_ref, dv8_ref, b2w_ref, out_ref):
    q = aggw_ref[0] + aggw_ref[1] + zsw_ref[...]
    out_ref[...] = dv8_ref[...] * q + b2w_ref[...]


_RB = 2048
_DSPEC = pl.BlockSpec((NC, NP), lambda i: (0, 0))


_R2 = _RB // 2


def _stage1(xw, W1b, degp):
    g = (NN + _RB - 1) // _RB
    return pl.pallas_call(
        _stage1_body,
        grid=(g,),
        in_specs=[
            pl.BlockSpec((_R2, 2 * DF), lambda i: (i, 0)),
            pl.BlockSpec((2 * DF, 2 * DH), lambda i: (0, 0)),
            _DSPEC,
        ],
        out_specs=[
            pl.BlockSpec((_R2, 2 * DH), lambda i: (i, 0)),
            pl.BlockSpec((_R2, 2 * DH), lambda i: (i, 0)),
            pl.BlockSpec((_PB, 128), lambda i: (i, 0)),
        ],
        out_shape=[
            jax.ShapeDtypeStruct((NN // 2, 2 * DH), jnp.float32),
            jax.ShapeDtypeStruct((NN // 2, 2 * DH), jnp.float32),
            jax.ShapeDtypeStruct((NN * DO // 128, 128), jnp.float32),
        ],
    )(xw, W1b, degp)


def _stage2(agg1w, hsw, dvw, b1w, W2b):
    g = (NN + _RB - 1) // _RB
    return pl.pallas_call(
        _stage2_body,
        grid=(g,),
        in_specs=[
            pl.BlockSpec((NC, _R2, 2 * DH), lambda i: (0, i, 0)),
            pl.BlockSpec((_R2, 2 * DH), lambda i: (i, 0)),
            pl.BlockSpec((_R2, 2 * DH), lambda i: (i, 0)),
            pl.BlockSpec((1, 2 * DH), lambda i: (0, 0)),
            pl.BlockSpec((2 * DH, 2 * DO), lambda i: (0, 0)),
        ],
        out_specs=pl.BlockSpec((_R2, 2 * DO), lambda i: (i, 0)),
        out_shape=jax.ShapeDtypeStruct((NN // 2, 2 * DO), jnp.float32),
    )(agg1w, hsw, dvw, b1w, W2b)


_PB = _RB // _NPR


def _stage3(agg2w, zsw, dv8, b2w):
    g = (NN + _RB - 1) // _RB
    return pl.pallas_call(
        _stage3_body,
        grid=(g,),
        in_specs=[
            pl.BlockSpec((NC, _PB, 128), lambda i: (0, i, 0)),
            pl.BlockSpec((_PB, 128), lambda i: (i, 0)),
            pl.BlockSpec((_PB, 128), lambda i: (i, 0)),
            pl.BlockSpec((1, 128), lambda i: (0, 0)),
        ],
        out_specs=pl.BlockSpec((_PB, 128), lambda i: (i, 0)),
        out_shape=jax.ShapeDtypeStruct((NN * DO // 128, 128), jnp.float32),
    )(agg2w, zsw, dv8, b2w)


def _blockdiag2(W):
    r, c = W.shape
    Wb = jnp.zeros((2 * r, 2 * c), W.dtype)
    return Wb.at[:r, :c].set(W).at[r:, c:].set(W)


def kernel(x, edge_index, W1, b1, W2, b2):
    ei3 = edge_index.astype(jnp.int32).reshape(2, NW, NCH, CHUNK)
    xw = x.reshape(NN // 2, 2 * DF)
    W1b = _blockdiag2(W1)
    W2p = jnp.pad(W2, ((0, 0), (0, DO - W2.shape[1])))
    W2b = _blockdiag2(W2p)
    b1w = jnp.tile(b1.reshape(1, DH), (1, 2))
    b2w = jnp.tile(jnp.pad(b2, (0, DO - b2.shape[0])).reshape(1, DO),
                   (1, _NPR))
    z64 = jnp.zeros((SPT, DH), jnp.float32)
    z16 = jnp.zeros((SPT, DO), jnp.float32)

    degp = _deg_kernel(ei3)
    hsw, dvw, dv8 = _stage1(xw, W1b, degp)
    hs = hsw.reshape(NN, DH)
    agg1 = _agg64(ei3, hs, z64)
    agg1w = agg1.reshape(NC, NP * DH // 128, 128)
    zs = _stage2(agg1w, hsw, dvw, b1w, W2b)
    zs16 = zs.reshape(NN, DO)
    agg2 = _agg16(ei3, zs16, z16)
    agg2w = agg2.reshape(NC, NP * DO // 128, 128)
    zsw = zs.reshape(NN * DO // 128, 128)
    outw = _stage3(agg2w, zsw, dv8, b2w)
    return outw.reshape(NN, DO)[:, :8]
